# baseline (speedup 1.0000x reference)
import jax
import jax.numpy as jnp
from jax import lax
from jax.experimental import pallas as pl
from jax.experimental.pallas import tpu as pltpu
from jax.experimental.pallas import tpu_sc as plsc

N_NODES = 10000
N_EDGES = 320000
C = 128
NC = 2
NS = 16
NW = NC * NS
K = 80
NCHUNK = N_EDGES // K
NCHUNK_CORE = NCHUNK // NC
NCH_TILE = NCHUNK_CORE // NS
NR = 624
NR_LAST_EXTRA = N_NODES - NS * NR
SHIFT = 14


def _transpose_body(x_ref, o_ref):
    o_ref[...] = x_ref[...].T


def _transpose(x2d):
    return pl.pallas_call(
        _transpose_body,
        out_shape=jax.ShapeDtypeStruct((N_NODES, C), jnp.float32),
    )(x2d)


def _pack_body(i_ref, j_ref, o_ref):
    o_ref[...] = (i_ref[...] << SHIFT) | j_ref[...]


def _pack(iInd, jInd):
    return pl.pallas_call(
        _pack_body,
        out_shape=jax.ShapeDtypeStruct((NCHUNK, K), jnp.int32),
    )(iInd.reshape(NCHUNK, K), jInd.reshape(NCHUNK, K)).reshape(
        NC, N_EDGES // NC)


def _edge_kernel_body(xT, Wh, pk, om_hbm, osum_hbm, w_v, s_v, om, pk_all,
                      ii0, ii1, jj0, jj1, r0, r1,
                      g0, g1, s0, s1):
    cid = lax.axis_index("c")
    sid = lax.axis_index("s")

    zero16 = jnp.zeros((16,), jnp.float32)

    def zrow(r, carry):
        for v in range(C // 16):
            r0[r, pl.ds(v * 16, 16)] = zero16
        return carry

    lax.fori_loop(0, K, zrow, 0)

    def zs(r, carry):
        s_v[pl.ds(r * 16, 16)] = zero16
        return carry

    lax.fori_loop(0, N_NODES // 16, zs, 0)

    nbase = sid * NR
    def zacc(t, carry):
        pltpu.sync_copy(r0.at[pl.ds(0, 78)],
                        om.at[pl.ds(nbase + t * 78, 78)])
        return carry
    lax.fori_loop(0, NR // 78, zacc, 0)

    @pl.when(sid == NS - 1)
    def _zero_tail():
        pltpu.sync_copy(r0.at[pl.ds(0, NR_LAST_EXTRA)],
                        om.at[pl.ds(NS * NR, NR_LAST_EXTRA)])

    pltpu.sync_copy(pk.at[cid, pl.ds(sid * NCH_TILE * K, NCH_TILE * K)],
                    pk_all)
    pltpu.sync_copy(Wh, w_v)
    plsc.subcore_barrier()

    def issue(t, iib, jjb, rb, gb):
        def unpack(g, c2):
            v = pk_all[pl.ds(t * K + g * 16, 16)]
            iib[pl.ds(g * 16, 16)] = v >> SHIFT
            jjb[pl.ds(g * 16, 16)] = v & ((1 << SHIFT) - 1)
            return c2

        lax.fori_loop(0, K // 16, unpack, 0)
        pltpu.async_copy(xT.at[jjb], rb, gb)

    def process(iib, jjb, rb, gb, sb):
        pltpu.make_async_copy(xT.at[pl.ds(0, K)], rb, gb).wait()

        def grp(g, c2):
            vi = iib[pl.ds(g * 16, 16)]
            vj = jjb[pl.ds(g * 16, 16)]
            wi = plsc.load_gather(w_v, [vi])
            wj = plsc.load_gather(w_v, [vj])
            cv = wi * (wi + wj)
            plsc.addupdate_scatter(s_v, [vi], cv)
            for k in range(16):
                e = g * 16 + k
                cs = cv[k]
                for v in range(C // 16):
                    rb[e, pl.ds(v * 16, 16)] = rb[e, pl.ds(v * 16, 16)] * cs
            return c2

        lax.fori_loop(0, K // 16, grp, 0)
        pltpu.async_copy(rb, om.at[iib], sb, add=True)

    def drain_scatter(iib, rb, sb):
        pltpu.make_async_copy(rb, om.at[iib], sb).wait()

    issue(0, ii0, jj0, r0, g0)

    def pair_body(tp, carry):
        t0 = tp * 2

        @pl.when(t0 + 1 < NCH_TILE)
        def _i1():
            @pl.when(t0 >= 1)
            def _d1():
                drain_scatter(ii1, r1, s1)
            issue(t0 + 1, ii1, jj1, r1, g1)

        process(ii0, jj0, r0, g0, s0)

        @pl.when(t0 + 2 < NCH_TILE)
        def _i2():
            drain_scatter(ii0, r0, s0)
            issue(t0 + 2, ii0, jj0, r0, g0)

        @pl.when(t0 + 1 < NCH_TILE)
        def _p2():
            process(ii1, jj1, r1, g1, s1)

        return carry

    lax.fori_loop(0, (NCH_TILE + 1) // 2, pair_body, 0)
    drain_scatter(ii0, r0, s0)
    drain_scatter(ii1, r1, s1)
    plsc.subcore_barrier()

    pltpu.sync_copy(om.at[pl.ds(nbase, NR)],
                    om_hbm.at[cid, pl.ds(nbase, NR)])

    @pl.when(sid == NS - 1)
    def _tail():
        pltpu.sync_copy(om.at[pl.ds(NS * NR, NR_LAST_EXTRA)],
                        om_hbm.at[cid, pl.ds(NS * NR, NR_LAST_EXTRA)])

    wid = cid * NS + sid
    pltpu.sync_copy(s_v, osum_hbm.at[pl.ds(wid * N_NODES, N_NODES)])


def _edge_scatter(xT, W, packed):
    mesh = plsc.VectorSubcoreMesh(core_axis_name="c", subcore_axis_name="s",
                                  num_cores=NC, num_subcores=NS)
    f = pl.kernel(
        _edge_kernel_body,
        out_type=(jax.ShapeDtypeStruct((NC, N_NODES, C), jnp.float32),
                  jax.ShapeDtypeStruct((NW * N_NODES,), jnp.float32)),
        mesh=mesh,
        compiler_params=pltpu.CompilerParams(needs_layout_passes=False,
                                             use_tc_tiling_on_sc=False),
        scratch_types=[
            pltpu.VMEM((N_NODES,), jnp.float32),
            pltpu.VMEM((N_NODES,), jnp.float32),
            pltpu.VMEM_SHARED((N_NODES, C), jnp.float32),
            pltpu.VMEM((NCH_TILE * K,), jnp.int32),
            pltpu.VMEM((K,), jnp.int32),
            pltpu.VMEM((K,), jnp.int32),
            pltpu.VMEM((K,), jnp.int32),
            pltpu.VMEM((K,), jnp.int32),
            pltpu.VMEM((K, C), jnp.float32),
            pltpu.VMEM((K, C), jnp.float32),
            pltpu.SemaphoreType.DMA,
            pltpu.SemaphoreType.DMA,
            pltpu.SemaphoreType.DMA,
            pltpu.SemaphoreType.DMA,
        ],
    )
    return f(xT, W, packed)


def _combine_body(x_ref, am_ref, as_ref, o_ref):
    s = jnp.sum(as_ref[...], axis=0, keepdims=True)
    a = am_ref[0] + am_ref[1]
    o_ref[...] = x_ref[...] * s - a.T


def _combine(x2d, om, osum):
    return pl.pallas_call(
        _combine_body,
        out_shape=jax.ShapeDtypeStruct((C, N_NODES), jnp.float32),
    )(x2d, om, osum.reshape(NW, N_NODES))


def kernel(x, W, iInd, jInd):
    x2d = x[0]
    xT = _transpose(x2d)
    packed = _pack(iInd.astype(jnp.int32), jInd.astype(jnp.int32))
    om, osum = _edge_scatter(xT, W, packed)
    out2d = _combine(x2d, om, osum)
    return out2d[None]


# speedup vs baseline: 1.0719x; 1.0719x over previous
import jax
import jax.numpy as jnp
from jax import lax
from jax.experimental import pallas as pl
from jax.experimental.pallas import tpu as pltpu
from jax.experimental.pallas import tpu_sc as plsc

N_NODES = 10000
N_EDGES = 320000
C = 128
NC = 2
NS = 16
NW = NC * NS
K = 64
NCHUNK = N_EDGES // K
NCHUNK_CORE = NCHUNK // NC
NR = 624
NR_LAST_EXTRA = N_NODES - NS * NR
SHIFT = 14


def _transpose_body(x_ref, o_ref):
    o_ref[...] = x_ref[...].T


def _transpose(x2d):
    return pl.pallas_call(
        _transpose_body,
        out_shape=jax.ShapeDtypeStruct((N_NODES, C), jnp.float32),
    )(x2d)


def _pack_body(i_ref, j_ref, o_ref):
    o_ref[...] = (i_ref[...] << SHIFT) | j_ref[...]


def _pack(iInd, jInd):
    return pl.pallas_call(
        _pack_body,
        out_shape=jax.ShapeDtypeStruct((NCHUNK, K), jnp.int32),
    )(iInd.reshape(NCHUNK, K), jInd.reshape(NCHUNK, K)).reshape(
        NC, N_EDGES // NC)


def _edge_kernel_body(xT, Wh, pk, om_hbm, osum_hbm, w_v, s_v, om,
                      ii0, ii1, ii2, jj0, jj1, jj2, r0, r1, r2,
                      pk0, pk1, pk2, g0, g1, g2, s0, s1, s2, p0, p1, p2):
    cid = lax.axis_index("c")
    sid = lax.axis_index("s")

    zero16 = jnp.zeros((16,), jnp.float32)

    def zrow(r, carry):
        for v in range(C // 16):
            r0[r, pl.ds(v * 16, 16)] = zero16
        return carry

    lax.fori_loop(0, K, zrow, 0)

    def zs(r, carry):
        s_v[pl.ds(r * 16, 16)] = zero16
        return carry

    lax.fori_loop(0, N_NODES // 16, zs, 0)

    nbase = sid * NR
    def zacc(t, carry):
        pltpu.sync_copy(r0.at[pl.ds(0, 52)],
                        om.at[pl.ds(nbase + t * 52, 52)])
        return carry
    lax.fori_loop(0, NR // 52, zacc, 0)

    @pl.when(sid == NS - 1)
    def _zero_tail():
        pltpu.sync_copy(r0.at[pl.ds(0, NR_LAST_EXTRA)],
                        om.at[pl.ds(NS * NR, NR_LAST_EXTRA)])

    pltpu.sync_copy(Wh, w_v)
    plsc.subcore_barrier()

    nch = (NCHUNK_CORE - sid + NS - 1) // NS

    def pk_fetch(t, pkb, pb):
        base = (sid + t * NS) * K
        pltpu.async_copy(pk.at[cid, pl.ds(base, K)], pkb, pb)

    def issue(t, iib, jjb, rb, gb, pkb, pb):
        pltpu.make_async_copy(pk.at[cid, pl.ds(0, K)], pkb, pb).wait()

        def unpack(g, c2):
            v = pkb[pl.ds(g * 16, 16)]
            iib[pl.ds(g * 16, 16)] = v >> SHIFT
            jjb[pl.ds(g * 16, 16)] = v & ((1 << SHIFT) - 1)
            return c2

        lax.fori_loop(0, K // 16, unpack, 0)
        pltpu.async_copy(xT.at[jjb], rb, gb)

        @pl.when(t + 3 < nch)
        def _prefetch():
            pk_fetch(t + 3, pkb, pb)

    def process(iib, jjb, rb, gb, sb):
        pltpu.make_async_copy(xT.at[pl.ds(0, K)], rb, gb).wait()

        def grp(g, c2):
            vi = iib[pl.ds(g * 16, 16)]
            vj = jjb[pl.ds(g * 16, 16)]
            wi = plsc.load_gather(w_v, [vi])
            wj = plsc.load_gather(w_v, [vj])
            cv = wi * (wi + wj)
            plsc.addupdate_scatter(s_v, [vi], cv)
            for k in range(16):
                e = g * 16 + k
                cs = cv[k]
                for v in range(C // 16):
                    rb[e, pl.ds(v * 16, 16)] = rb[e, pl.ds(v * 16, 16)] * cs
            return c2

        lax.fori_loop(0, K // 16, grp, 0)
        pltpu.async_copy(rb, om.at[iib], sb, add=True)

    def drain_scatter(iib, rb, sb):
        pltpu.make_async_copy(rb, om.at[iib], sb).wait()

    slots = ((ii0, jj0, r0, g0, pk0, p0, s0),
             (ii1, jj1, r1, g1, pk1, p1, s1),
             (ii2, jj2, r2, g2, pk2, p2, s2))

    def issue_slot(t, sl):
        issue(t, sl[0], sl[1], sl[2], sl[3], sl[4], sl[5])

    def process_slot(sl):
        process(sl[0], sl[1], sl[2], sl[3], sl[6])

    def drain_slot(sl):
        drain_scatter(sl[0], sl[2], sl[6])

    pk_fetch(0, pk0, p0)
    pk_fetch(1, pk1, p1)
    pk_fetch(2, pk2, p2)
    issue_slot(0, slots[0])
    issue_slot(1, slots[1])

    def triple_body(tq, carry):
        t0 = tq * 3
        for r in range(3):
            t = t0 + r

            @pl.when(t < nch)
            def _p():
                process_slot(slots[r])

            @pl.when(t + 2 < nch)
            def _i():
                @pl.when(t >= 1)
                def _d():
                    drain_slot(slots[(r + 2) % 3])
                issue_slot(t + 2, slots[(r + 2) % 3])

        return carry

    lax.fori_loop(0, (nch + 2) // 3, triple_body, 0)
    drain_slot(slots[0])
    drain_slot(slots[1])
    drain_slot(slots[2])
    plsc.subcore_barrier()

    pltpu.sync_copy(om.at[pl.ds(nbase, NR)],
                    om_hbm.at[cid, pl.ds(nbase, NR)])

    @pl.when(sid == NS - 1)
    def _tail():
        pltpu.sync_copy(om.at[pl.ds(NS * NR, NR_LAST_EXTRA)],
                        om_hbm.at[cid, pl.ds(NS * NR, NR_LAST_EXTRA)])

    wid = cid * NS + sid
    pltpu.sync_copy(s_v, osum_hbm.at[pl.ds(wid * N_NODES, N_NODES)])


def _edge_scatter(xT, W, packed):
    mesh = plsc.VectorSubcoreMesh(core_axis_name="c", subcore_axis_name="s",
                                  num_cores=NC, num_subcores=NS)
    f = pl.kernel(
        _edge_kernel_body,
        out_type=(jax.ShapeDtypeStruct((NC, N_NODES, C), jnp.float32),
                  jax.ShapeDtypeStruct((NW * N_NODES,), jnp.float32)),
        mesh=mesh,
        compiler_params=pltpu.CompilerParams(needs_layout_passes=False,
                                             use_tc_tiling_on_sc=False),
        scratch_types=[
            pltpu.VMEM((N_NODES,), jnp.float32),
            pltpu.VMEM((N_NODES,), jnp.float32),
            pltpu.VMEM_SHARED((N_NODES, C), jnp.float32),
            pltpu.VMEM((K,), jnp.int32),
            pltpu.VMEM((K,), jnp.int32),
            pltpu.VMEM((K,), jnp.int32),
            pltpu.VMEM((K,), jnp.int32),
            pltpu.VMEM((K,), jnp.int32),
            pltpu.VMEM((K,), jnp.int32),
            pltpu.VMEM((K, C), jnp.float32),
            pltpu.VMEM((K, C), jnp.float32),
            pltpu.VMEM((K, C), jnp.float32),
            pltpu.VMEM((K,), jnp.int32),
            pltpu.VMEM((K,), jnp.int32),
            pltpu.VMEM((K,), jnp.int32),
            pltpu.SemaphoreType.DMA,
            pltpu.SemaphoreType.DMA,
            pltpu.SemaphoreType.DMA,
            pltpu.SemaphoreType.DMA,
            pltpu.SemaphoreType.DMA,
            pltpu.SemaphoreType.DMA,
            pltpu.SemaphoreType.DMA,
            pltpu.SemaphoreType.DMA,
            pltpu.SemaphoreType.DMA,
        ],
    )
    return f(xT, W, packed)


def _combine_body(x_ref, am_ref, as_ref, o_ref):
    s = jnp.sum(as_ref[...], axis=0, keepdims=True)
    a = am_ref[0] + am_ref[1]
    o_ref[...] = x_ref[...] * s - a.T


def _combine(x2d, om, osum):
    return pl.pallas_call(
        _combine_body,
        out_shape=jax.ShapeDtypeStruct((C, N_NODES), jnp.float32),
    )(x2d, om, osum.reshape(NW, N_NODES))


def kernel(x, W, iInd, jInd):
    x2d = x[0]
    xT = _transpose(x2d)
    packed = _pack(iInd.astype(jnp.int32), jInd.astype(jnp.int32))
    om, osum = _edge_scatter(xT, W, packed)
    out2d = _combine(x2d, om, osum)
    return out2d[None]


# speedup vs baseline: 1.0927x; 1.0193x over previous
import jax
import jax.numpy as jnp
from jax import lax
from jax.experimental import pallas as pl
from jax.experimental.pallas import tpu as pltpu
from jax.experimental.pallas import tpu_sc as plsc

N_NODES = 10000
N_EDGES = 320000
C = 128
NC = 2
NS = 16
NW = NC * NS
K = 64
NCHUNK = N_EDGES // K
NCHUNK_CORE = NCHUNK // NC
NR = 624
NR_LAST_EXTRA = N_NODES - NS * NR
SHIFT = 14


def _prep_body(x_ref, i_ref, j_ref, xt_ref, pk_ref):
    xt_ref[...] = x_ref[...].T
    pk_ref[...] = (i_ref[...] << SHIFT) | j_ref[...]


def _prep(x2d, iInd, jInd):
    xT, pk = pl.pallas_call(
        _prep_body,
        out_shape=(jax.ShapeDtypeStruct((N_NODES, C), jnp.float32),
                   jax.ShapeDtypeStruct((NCHUNK, K), jnp.int32)),
    )(x2d, iInd.reshape(NCHUNK, K), jInd.reshape(NCHUNK, K))
    return xT, pk.reshape(NC, N_EDGES // NC)


def _edge_kernel_body(xT, Wh, pk, om_hbm, osum_hbm, w_v, s_v, om,
                      ii0, ii1, ii2, jj0, jj1, jj2, r0, r1, r2,
                      pk0, pk1, pk2, g0, g1, g2, s0, s1, s2, p0, p1, p2):
    cid = lax.axis_index("c")
    sid = lax.axis_index("s")

    zero16 = jnp.zeros((16,), jnp.float32)

    def zrow(r, carry):
        for v in range(C // 16):
            r0[r, pl.ds(v * 16, 16)] = zero16
        return carry

    lax.fori_loop(0, K, zrow, 0)

    def zs(r, carry):
        for u in range(5):
            s_v[pl.ds((r * 5 + u) * 16, 16)] = zero16
        return carry

    lax.fori_loop(0, N_NODES // 80, zs, 0)

    nbase = sid * NR
    def zacc(t, carry):
        pltpu.async_copy(r0.at[pl.ds(0, 52)],
                         om.at[pl.ds(nbase + t * 52, 52)], g0)
        return carry
    lax.fori_loop(0, NR // 52, zacc, 0)

    def zacc_wait(t, carry):
        pltpu.make_async_copy(r0.at[pl.ds(0, 52)],
                              om.at[pl.ds(nbase, 52)], g0).wait()
        return carry
    lax.fori_loop(0, NR // 52, zacc_wait, 0)

    @pl.when(sid == NS - 1)
    def _zero_tail():
        pltpu.sync_copy(r0.at[pl.ds(0, NR_LAST_EXTRA)],
                        om.at[pl.ds(NS * NR, NR_LAST_EXTRA)])

    pltpu.sync_copy(Wh, w_v)
    plsc.subcore_barrier()

    nch = (NCHUNK_CORE - sid + NS - 1) // NS

    def pk_fetch(t, pkb, pb):
        base = (sid + t * NS) * K
        pltpu.async_copy(pk.at[cid, pl.ds(base, K)], pkb, pb)

    def issue(t, iib, jjb, rb, gb, pkb, pb):
        pltpu.make_async_copy(pk.at[cid, pl.ds(0, K)], pkb, pb).wait()

        def unpack(g, c2):
            v = pkb[pl.ds(g * 16, 16)]
            iib[pl.ds(g * 16, 16)] = v >> SHIFT
            jjb[pl.ds(g * 16, 16)] = v & ((1 << SHIFT) - 1)
            return c2

        lax.fori_loop(0, K // 16, unpack, 0)
        pltpu.async_copy(xT.at[jjb], rb, gb)

        @pl.when(t + 3 < nch)
        def _prefetch():
            pk_fetch(t + 3, pkb, pb)

    def process(iib, jjb, rb, gb, sb):
        pltpu.make_async_copy(xT.at[pl.ds(0, K)], rb, gb).wait()

        def grp(g, c2):
            vi = iib[pl.ds(g * 16, 16)]
            vj = jjb[pl.ds(g * 16, 16)]
            wi = plsc.load_gather(w_v, [vi])
            wj = plsc.load_gather(w_v, [vj])
            cv = wi * (wi + wj)
            plsc.addupdate_scatter(s_v, [vi], cv)
            for k in range(16):
                e = g * 16 + k
                cs = cv[k]
                for v in range(C // 16):
                    rb[e, pl.ds(v * 16, 16)] = rb[e, pl.ds(v * 16, 16)] * cs
            return c2

        lax.fori_loop(0, K // 16, grp, 0)
        pltpu.async_copy(rb, om.at[iib], sb, add=True)

    def drain_scatter(iib, rb, sb):
        pltpu.make_async_copy(rb, om.at[iib], sb).wait()

    slots = ((ii0, jj0, r0, g0, pk0, p0, s0),
             (ii1, jj1, r1, g1, pk1, p1, s1),
             (ii2, jj2, r2, g2, pk2, p2, s2))

    def issue_slot(t, sl):
        issue(t, sl[0], sl[1], sl[2], sl[3], sl[4], sl[5])

    def process_slot(sl):
        process(sl[0], sl[1], sl[2], sl[3], sl[6])

    def drain_slot(sl):
        drain_scatter(sl[0], sl[2], sl[6])

    pk_fetch(0, pk0, p0)
    pk_fetch(1, pk1, p1)
    pk_fetch(2, pk2, p2)
    issue_slot(0, slots[0])
    issue_slot(1, slots[1])

    def triple_body(tq, carry):
        t0 = tq * 3
        for r in range(3):
            t = t0 + r

            @pl.when(t < nch)
            def _p():
                process_slot(slots[r])

            @pl.when(t + 2 < nch)
            def _i():
                @pl.when(t >= 1)
                def _d():
                    drain_slot(slots[(r + 2) % 3])
                issue_slot(t + 2, slots[(r + 2) % 3])

        return carry

    lax.fori_loop(0, (nch + 2) // 3, triple_body, 0)
    drain_slot(slots[0])
    drain_slot(slots[1])
    drain_slot(slots[2])
    plsc.subcore_barrier()

    pltpu.sync_copy(om.at[pl.ds(nbase, NR)],
                    om_hbm.at[cid, pl.ds(nbase, NR)])

    @pl.when(sid == NS - 1)
    def _tail():
        pltpu.sync_copy(om.at[pl.ds(NS * NR, NR_LAST_EXTRA)],
                        om_hbm.at[cid, pl.ds(NS * NR, NR_LAST_EXTRA)])

    wid = cid * NS + sid
    pltpu.sync_copy(s_v, osum_hbm.at[pl.ds(wid * N_NODES, N_NODES)])


def _edge_scatter(xT, W, packed):
    mesh = plsc.VectorSubcoreMesh(core_axis_name="c", subcore_axis_name="s",
                                  num_cores=NC, num_subcores=NS)
    f = pl.kernel(
        _edge_kernel_body,
        out_type=(jax.ShapeDtypeStruct((NC, N_NODES, C), jnp.float32),
                  jax.ShapeDtypeStruct((NW * N_NODES,), jnp.float32)),
        mesh=mesh,
        compiler_params=pltpu.CompilerParams(needs_layout_passes=False,
                                             use_tc_tiling_on_sc=False),
        scratch_types=[
            pltpu.VMEM((N_NODES,), jnp.float32),
            pltpu.VMEM((N_NODES,), jnp.float32),
            pltpu.VMEM_SHARED((N_NODES, C), jnp.float32),
            pltpu.VMEM((K,), jnp.int32),
            pltpu.VMEM((K,), jnp.int32),
            pltpu.VMEM((K,), jnp.int32),
            pltpu.VMEM((K,), jnp.int32),
            pltpu.VMEM((K,), jnp.int32),
            pltpu.VMEM((K,), jnp.int32),
            pltpu.VMEM((K, C), jnp.float32),
            pltpu.VMEM((K, C), jnp.float32),
            pltpu.VMEM((K, C), jnp.float32),
            pltpu.VMEM((K,), jnp.int32),
            pltpu.VMEM((K,), jnp.int32),
            pltpu.VMEM((K,), jnp.int32),
            pltpu.SemaphoreType.DMA,
            pltpu.SemaphoreType.DMA,
            pltpu.SemaphoreType.DMA,
            pltpu.SemaphoreType.DMA,
            pltpu.SemaphoreType.DMA,
            pltpu.SemaphoreType.DMA,
            pltpu.SemaphoreType.DMA,
            pltpu.SemaphoreType.DMA,
            pltpu.SemaphoreType.DMA,
        ],
    )
    return f(xT, W, packed)


def _combine_body(x_ref, am_ref, as_ref, o_ref):
    s = jnp.sum(as_ref[...], axis=0, keepdims=True)
    a = am_ref[0] + am_ref[1]
    o_ref[...] = x_ref[...] * s - a.T


def _combine(x2d, om, osum):
    return pl.pallas_call(
        _combine_body,
        out_shape=jax.ShapeDtypeStruct((C, N_NODES), jnp.float32),
    )(x2d, om, osum.reshape(NW, N_NODES))


def kernel(x, W, iInd, jInd):
    x2d = x[0]
    xT, packed = _prep(x2d, iInd.astype(jnp.int32), jInd.astype(jnp.int32))
    om, osum = _edge_scatter(xT, W, packed)
    out2d = _combine(x2d, om, osum)
    return out2d[None]
